# Initial kernel scaffold; baseline (speedup 1.0000x reference)
import jax
import jax.numpy as jnp
from jax.experimental import pallas as pl


def kernel(z_e, embedding):
    raise NotImplementedError("write your pallas kernel here")


# speedup vs baseline: 1.5581x; 1.5581x over previous
import functools

import jax
import jax.numpy as jnp
from jax.experimental import pallas as pl


def _vq_block_kernel(ze_ref, emb_ref, zq_ref, idx_ref, cnt_ref, loss_ref, *, num_codes):
    i = pl.program_id(0)
    ze = ze_ref[0]
    emb = emb_ref[...]

    e_sq = jnp.sum(emb * emb, axis=1)
    x_sq = jnp.sum(ze * ze, axis=0)
    dot = jax.lax.dot_general(
        ze.astype(jnp.bfloat16), emb.astype(jnp.bfloat16), (((0,), (1,)), ((), ())),
        preferred_element_type=jnp.float32,
    )
    dist = (x_sq[:, None] - 2.0 * dot) + e_sq[None, :]

    rowmin = jnp.min(dist, axis=1, keepdims=True)
    iota = jax.lax.broadcasted_iota(jnp.int32, dist.shape, 1)
    idx = jnp.min(jnp.where(dist == rowmin, iota, num_codes), axis=1)

    onehot = (iota == idx[:, None]).astype(jnp.float32)
    zq = jax.lax.dot_general(
        emb, onehot, (((0,), (1,)), ((), ())),
        precision=jax.lax.Precision.HIGHEST,
        preferred_element_type=jnp.float32,
    )

    zq_ref[0] = ze + (zq - ze)
    idx_ref[0, 0, :] = idx

    blk_counts = jnp.sum(onehot, axis=0)
    blk_loss = jnp.full((128,), jnp.sum((ze - zq) ** 2), jnp.float32)

    @pl.when(i == 0)
    def _init():
        cnt_ref[0, :] = blk_counts
        loss_ref[0, :] = blk_loss

    @pl.when(i > 0)
    def _accum():
        cnt_ref[0, :] += blk_counts
        loss_ref[0, :] += blk_loss


def kernel(z_e, embedding):
    B, d, T = z_e.shape
    K = embedding.shape[0]
    Tblk = 2048
    tpb = T // Tblk
    grid = B * tpb

    zq_st, idx3, counts2, loss2 = pl.pallas_call(
        functools.partial(_vq_block_kernel, num_codes=K),
        grid=(grid,),
        in_specs=[
            pl.BlockSpec((1, d, Tblk), lambda i: (i // tpb, 0, i % tpb)),
            pl.BlockSpec((K, d), lambda i: (0, 0)),
        ],
        out_specs=[
            pl.BlockSpec((1, d, Tblk), lambda i: (i // tpb, 0, i % tpb)),
            pl.BlockSpec((1, 1, Tblk), lambda i: (i, 0, 0)),
            pl.BlockSpec((1, K), lambda i: (0, 0)),
            pl.BlockSpec((1, 128), lambda i: (0, 0)),
        ],
        out_shape=[
            jax.ShapeDtypeStruct((B, d, T), jnp.float32),
            jax.ShapeDtypeStruct((grid, 1, Tblk), jnp.int32),
            jax.ShapeDtypeStruct((1, K), jnp.float32),
            jax.ShapeDtypeStruct((1, 128), jnp.float32),
        ],
    )(z_e, embedding)

    indices = idx3.reshape(B, T)
    counts = counts2.reshape(K)
    commit_loss = 0.25 * (loss2[0, 0] / (B * d * T))
    probs = counts / jnp.maximum(counts.sum(), 1.0)
    perplexity = jnp.exp(-jnp.sum(probs * jnp.log(probs + 1e-10)))
    return (zq_st, commit_loss, indices, perplexity, counts)


# speedup vs baseline: 2.4285x; 1.5586x over previous
import functools

import jax
import jax.numpy as jnp
from jax.experimental import pallas as pl


def _vq_block_kernel(ze_ref, emb_ref, zq_ref, idx_ref, cnt_ref, loss_ref, *, num_codes):
    i = pl.program_id(0)
    ze = ze_ref[0]
    emb = emb_ref[...]
    tblk = ze.shape[1]

    e_sq = jnp.sum(emb * emb, axis=1)
    x_sq = jnp.sum(ze * ze, axis=0)
    dot = jax.lax.dot_general(
        ze.astype(jnp.bfloat16), emb.astype(jnp.bfloat16), (((0,), (1,)), ((), ())),
        preferred_element_type=jnp.float32,
    )
    dist = (x_sq[:, None] - 2.0 * dot) + e_sq[None, :]

    rowmin = jnp.min(dist, axis=1, keepdims=True)
    iotaf = jax.lax.broadcasted_iota(jnp.int32, dist.shape, 1).astype(jnp.float32)
    idxf = jnp.min(jnp.where(dist == rowmin, iotaf, jnp.float32(num_codes)),
                   axis=1)
    idx = idxf.astype(jnp.int32)

    onehot = (iotaf == idxf[:, None]).astype(jnp.bfloat16)
    emb_hi = emb.astype(jnp.bfloat16)
    emb_lo = (emb - emb_hi.astype(jnp.float32)).astype(jnp.bfloat16)
    zq_hi = jax.lax.dot_general(
        emb_hi, onehot, (((0,), (1,)), ((), ())),
        preferred_element_type=jnp.float32)
    zq_lo = jax.lax.dot_general(
        emb_lo, onehot, (((0,), (1,)), ((), ())),
        preferred_element_type=jnp.float32)
    zq = zq_hi + zq_lo

    zq_ref[0] = ze + (zq - ze)
    idx_ref[0, 0, :] = idx

    ones_row = jnp.ones((1, tblk), jnp.bfloat16)
    blk_counts = jax.lax.dot_general(
        ones_row, onehot, (((1,), (0,)), ((), ())),
        preferred_element_type=jnp.float32)
    blk_loss = jnp.full((128,), jnp.sum((ze - zq) ** 2), jnp.float32)

    @pl.when(i == 0)
    def _init():
        cnt_ref[0, :] = blk_counts[0]
        loss_ref[0, :] = blk_loss

    @pl.when(i > 0)
    def _accum():
        cnt_ref[0, :] += blk_counts[0]
        loss_ref[0, :] += blk_loss


def kernel(z_e, embedding):
    B, d, T = z_e.shape
    K = embedding.shape[0]
    Tblk = 2048
    tpb = T // Tblk
    grid = B * tpb

    zq_st, idx3, counts2, loss2 = pl.pallas_call(
        functools.partial(_vq_block_kernel, num_codes=K),
        grid=(grid,),
        in_specs=[
            pl.BlockSpec((1, d, Tblk), lambda i: (i // tpb, 0, i % tpb)),
            pl.BlockSpec((K, d), lambda i: (0, 0)),
        ],
        out_specs=[
            pl.BlockSpec((1, d, Tblk), lambda i: (i // tpb, 0, i % tpb)),
            pl.BlockSpec((1, 1, Tblk), lambda i: (i, 0, 0)),
            pl.BlockSpec((1, K), lambda i: (0, 0)),
            pl.BlockSpec((1, 128), lambda i: (0, 0)),
        ],
        out_shape=[
            jax.ShapeDtypeStruct((B, d, T), jnp.float32),
            jax.ShapeDtypeStruct((grid, 1, Tblk), jnp.int32),
            jax.ShapeDtypeStruct((1, K), jnp.float32),
            jax.ShapeDtypeStruct((1, 128), jnp.float32),
        ],
    )(z_e, embedding)

    indices = idx3.reshape(B, T)
    counts = counts2.reshape(K)
    commit_loss = 0.25 * (loss2[0, 0] / (B * d * T))
    probs = counts / jnp.maximum(counts.sum(), 1.0)
    perplexity = jnp.exp(-jnp.sum(probs * jnp.log(probs + 1e-10)))
    return (zq_st, commit_loss, indices, perplexity, counts)


# speedup vs baseline: 3.4021x; 1.4009x over previous
import functools

import jax
import jax.numpy as jnp
from jax.experimental import pallas as pl


def _vq_block_kernel(ze_ref, emb_ref, zq_ref, idx_ref, cnt_ref, loss_ref, *, num_codes):
    i = pl.program_id(0)
    ze = ze_ref[0]
    emb = emb_ref[...]
    tblk = ze.shape[1]

    e_sq = jnp.sum(emb * emb, axis=1)
    x_sq = jnp.sum(ze * ze, axis=0)
    dot = jax.lax.dot_general(
        emb.astype(jnp.bfloat16), ze.astype(jnp.bfloat16), (((1,), (0,)), ((), ())),
        preferred_element_type=jnp.float32,
    )
    dist = (x_sq[None, :] - 2.0 * dot) + e_sq[:, None]

    colmin = jnp.min(dist, axis=0, keepdims=True)
    iotaf = jax.lax.broadcasted_iota(jnp.int32, (num_codes, 1), 0).astype(jnp.float32)
    idxf = jnp.min(jnp.where(dist == colmin, iotaf, jnp.float32(num_codes)),
                   axis=0)
    idx = idxf.astype(jnp.int32)

    onehot = (iotaf == idxf[None, :]).astype(jnp.bfloat16)
    emb_hi = emb.astype(jnp.bfloat16)
    emb_lo = (emb - emb_hi.astype(jnp.float32)).astype(jnp.bfloat16)
    zq_hi = jax.lax.dot_general(
        emb_hi, onehot, (((0,), (0,)), ((), ())),
        preferred_element_type=jnp.float32)
    zq_lo = jax.lax.dot_general(
        emb_lo, onehot, (((0,), (0,)), ((), ())),
        preferred_element_type=jnp.float32)
    zq = zq_hi + zq_lo

    zq_ref[0] = ze + (zq - ze)
    idx_ref[0, 0, :] = idx

    ones_col = jnp.ones((1, tblk), jnp.bfloat16)
    blk_counts = jax.lax.dot_general(
        ones_col, onehot, (((1,), (1,)), ((), ())),
        preferred_element_type=jnp.float32)
    blk_loss = jnp.full((128,), jnp.sum((ze - zq) ** 2), jnp.float32)

    @pl.when(i == 0)
    def _init():
        cnt_ref[0, :] = blk_counts[0]
        loss_ref[0, :] = blk_loss

    @pl.when(i > 0)
    def _accum():
        cnt_ref[0, :] += blk_counts[0]
        loss_ref[0, :] += blk_loss


def kernel(z_e, embedding):
    B, d, T = z_e.shape
    K = embedding.shape[0]
    Tblk = 2048
    tpb = T // Tblk
    grid = B * tpb

    zq_st, idx3, counts2, loss2 = pl.pallas_call(
        functools.partial(_vq_block_kernel, num_codes=K),
        grid=(grid,),
        in_specs=[
            pl.BlockSpec((1, d, Tblk), lambda i: (i // tpb, 0, i % tpb)),
            pl.BlockSpec((K, d), lambda i: (0, 0)),
        ],
        out_specs=[
            pl.BlockSpec((1, d, Tblk), lambda i: (i // tpb, 0, i % tpb)),
            pl.BlockSpec((1, 1, Tblk), lambda i: (i, 0, 0)),
            pl.BlockSpec((1, K), lambda i: (0, 0)),
            pl.BlockSpec((1, 128), lambda i: (0, 0)),
        ],
        out_shape=[
            jax.ShapeDtypeStruct((B, d, T), jnp.float32),
            jax.ShapeDtypeStruct((grid, 1, Tblk), jnp.int32),
            jax.ShapeDtypeStruct((1, K), jnp.float32),
            jax.ShapeDtypeStruct((1, 128), jnp.float32),
        ],
    )(z_e, embedding)

    indices = idx3.reshape(B, T)
    counts = counts2.reshape(K)
    commit_loss = 0.25 * (loss2[0, 0] / (B * d * T))
    probs = counts / jnp.maximum(counts.sum(), 1.0)
    perplexity = jnp.exp(-jnp.sum(probs * jnp.log(probs + 1e-10)))
    return (zq_st, commit_loss, indices, perplexity, counts)


# speedup vs baseline: 3.9343x; 1.1564x over previous
import functools

import jax
import jax.numpy as jnp
from jax.experimental import pallas as pl


def _vq_block_kernel(ze_ref, emb_ref, ecat_ref, zq_ref, idx_ref, cnt_ref,
                     loss_ref, *, num_codes):
    i = pl.program_id(0)
    ze = ze_ref[0]
    emb = emb_ref[...]
    ecat = ecat_ref[...]
    d = ze.shape[0]
    tblk = ze.shape[1]

    e_sq = jnp.sum(emb * emb, axis=1)
    x_sq = jnp.sum(ze * ze, axis=0)
    dot = jax.lax.dot_general(
        ecat[:, :d], ze.astype(jnp.bfloat16), (((1,), (0,)), ((), ())),
        preferred_element_type=jnp.float32,
    )
    dist = (x_sq[None, :] - 2.0 * dot) + e_sq[:, None]

    colmin = jnp.min(dist, axis=0, keepdims=True)
    iotaf = jax.lax.broadcasted_iota(jnp.int32, (num_codes, 1), 0).astype(jnp.float32)
    idxf = jnp.min(jnp.where(dist == colmin, iotaf, jnp.float32(num_codes)),
                   axis=0)
    idx = idxf.astype(jnp.int32)

    onehot = (iotaf == idxf[None, :]).astype(jnp.bfloat16)
    zq2 = jax.lax.dot_general(
        ecat, onehot, (((0,), (0,)), ((), ())),
        preferred_element_type=jnp.float32)
    zq = zq2[:d] + zq2[d:]

    zq_ref[0] = ze + (zq - ze)
    idx_ref[0, 0, :] = idx

    ones_col = jnp.ones((1, tblk), jnp.bfloat16)
    blk_counts = jax.lax.dot_general(
        ones_col, onehot, (((1,), (1,)), ((), ())),
        preferred_element_type=jnp.float32)
    blk_loss = jnp.full((128,), jnp.sum((ze - zq) ** 2), jnp.float32)

    @pl.when(i == 0)
    def _init():
        cnt_ref[0, :] = blk_counts[0]
        loss_ref[0, :] = blk_loss

    @pl.when(i > 0)
    def _accum():
        cnt_ref[0, :] += blk_counts[0]
        loss_ref[0, :] += blk_loss


def kernel(z_e, embedding):
    B, d, T = z_e.shape
    K = embedding.shape[0]
    Tblk = 4096
    tpb = T // Tblk
    grid = B * tpb

    emb_hi = embedding.astype(jnp.bfloat16)
    emb_lo = (embedding - emb_hi.astype(jnp.float32)).astype(jnp.bfloat16)
    emb_cat = jnp.concatenate([emb_hi, emb_lo], axis=1)

    zq_st, idx3, counts2, loss2 = pl.pallas_call(
        functools.partial(_vq_block_kernel, num_codes=K),
        grid=(grid,),
        in_specs=[
            pl.BlockSpec((1, d, Tblk), lambda i: (i // tpb, 0, i % tpb)),
            pl.BlockSpec((K, d), lambda i: (0, 0)),
            pl.BlockSpec((K, 2 * d), lambda i: (0, 0)),
        ],
        out_specs=[
            pl.BlockSpec((1, d, Tblk), lambda i: (i // tpb, 0, i % tpb)),
            pl.BlockSpec((1, 1, Tblk), lambda i: (i, 0, 0)),
            pl.BlockSpec((1, K), lambda i: (0, 0)),
            pl.BlockSpec((1, 128), lambda i: (0, 0)),
        ],
        out_shape=[
            jax.ShapeDtypeStruct((B, d, T), jnp.float32),
            jax.ShapeDtypeStruct((grid, 1, Tblk), jnp.int32),
            jax.ShapeDtypeStruct((1, K), jnp.float32),
            jax.ShapeDtypeStruct((1, 128), jnp.float32),
        ],
    )(z_e, embedding, emb_cat)

    indices = idx3.reshape(B, T)
    counts = counts2.reshape(K)
    commit_loss = 0.25 * (loss2[0, 0] / (B * d * T))
    probs = counts / jnp.maximum(counts.sum(), 1.0)
    perplexity = jnp.exp(-jnp.sum(probs * jnp.log(probs + 1e-10)))
    return (zq_st, commit_loss, indices, perplexity, counts)


# speedup vs baseline: 4.0458x; 1.0283x over previous
import functools

import jax
import jax.numpy as jnp
from jax.experimental import pallas as pl


def _vq_block_kernel(ze_ref, emb_ref, ecat_ref, zq_ref, idx_ref, cnt_ref,
                     loss_ref, *, num_codes):
    i = pl.program_id(0)
    ze = ze_ref[0]
    emb = emb_ref[...]
    ecat = ecat_ref[...]
    d = ze.shape[0]
    tblk = ze.shape[1]

    e_sq = jnp.sum(emb * emb, axis=1)
    x_sq = jnp.sum(ze * ze, axis=0)
    dot = jax.lax.dot_general(
        ecat[:, :d], ze.astype(jnp.bfloat16), (((1,), (0,)), ((), ())),
        preferred_element_type=jnp.float32,
    )
    dist = (x_sq[None, :] - 2.0 * dot) + e_sq[:, None]

    colmin = jnp.min(dist, axis=0, keepdims=True)
    iotaf = jax.lax.broadcasted_iota(jnp.int32, (num_codes, 1), 0).astype(jnp.float32)
    idxf = jnp.min(jnp.where(dist == colmin, iotaf, jnp.float32(num_codes)),
                   axis=0)
    idx = idxf.astype(jnp.int32)

    onehot = (iotaf == idxf[None, :]).astype(jnp.bfloat16)
    zq2 = jax.lax.dot_general(
        ecat, onehot, (((0,), (0,)), ((), ())),
        preferred_element_type=jnp.float32)
    zq = zq2[:d] + zq2[d:]

    zq_ref[0] = ze + (zq - ze)
    idx_ref[0, 0, :] = idx

    ones_col = jnp.ones((1, tblk), jnp.bfloat16)
    blk_counts = jax.lax.dot_general(
        ones_col, onehot, (((1,), (1,)), ((), ())),
        preferred_element_type=jnp.float32)
    blk_loss = jnp.full((128,), jnp.sum((ze - zq) ** 2), jnp.float32)

    @pl.when(i == 0)
    def _init():
        cnt_ref[0, :] = blk_counts[0]
        loss_ref[0, :] = blk_loss

    @pl.when(i > 0)
    def _accum():
        cnt_ref[0, :] += blk_counts[0]
        loss_ref[0, :] += blk_loss


def kernel(z_e, embedding):
    B, d, T = z_e.shape
    K = embedding.shape[0]
    Tblk = 8192
    tpb = T // Tblk
    grid = B * tpb

    emb_hi = embedding.astype(jnp.bfloat16)
    emb_lo = (embedding - emb_hi.astype(jnp.float32)).astype(jnp.bfloat16)
    emb_cat = jnp.concatenate([emb_hi, emb_lo], axis=1)

    zq_st, idx3, counts2, loss2 = pl.pallas_call(
        functools.partial(_vq_block_kernel, num_codes=K),
        grid=(grid,),
        in_specs=[
            pl.BlockSpec((1, d, Tblk), lambda i: (i // tpb, 0, i % tpb)),
            pl.BlockSpec((K, d), lambda i: (0, 0)),
            pl.BlockSpec((K, 2 * d), lambda i: (0, 0)),
        ],
        out_specs=[
            pl.BlockSpec((1, d, Tblk), lambda i: (i // tpb, 0, i % tpb)),
            pl.BlockSpec((1, 1, Tblk), lambda i: (i, 0, 0)),
            pl.BlockSpec((1, K), lambda i: (0, 0)),
            pl.BlockSpec((1, 128), lambda i: (0, 0)),
        ],
        out_shape=[
            jax.ShapeDtypeStruct((B, d, T), jnp.float32),
            jax.ShapeDtypeStruct((grid, 1, Tblk), jnp.int32),
            jax.ShapeDtypeStruct((1, K), jnp.float32),
            jax.ShapeDtypeStruct((1, 128), jnp.float32),
        ],
    )(z_e, embedding, emb_cat)

    indices = idx3.reshape(B, T)
    counts = counts2.reshape(K)
    commit_loss = 0.25 * (loss2[0, 0] / (B * d * T))
    probs = counts / jnp.maximum(counts.sum(), 1.0)
    perplexity = jnp.exp(-jnp.sum(probs * jnp.log(probs + 1e-10)))
    return (zq_st, commit_loss, indices, perplexity, counts)


# speedup vs baseline: 4.4466x; 1.0991x over previous
import functools

import jax
import jax.numpy as jnp
from jax.experimental import pallas as pl


def _vq_block_kernel(ze_ref, emb_ref, ecat_ref, zq_ref, idx_ref, cnt_ref,
                     loss_ref, *, num_codes):
    i = pl.program_id(0)
    ze = ze_ref[0]
    emb = emb_ref[...]
    ecat = ecat_ref[...]
    d = ze.shape[0]
    tblk = ze.shape[1]

    e_sq = jnp.sum(emb * emb, axis=1)
    x_sq = jnp.sum(ze * ze, axis=0)
    dot = jax.lax.dot_general(
        ecat[:, :d], ze.astype(jnp.bfloat16), (((1,), (0,)), ((), ())),
        preferred_element_type=jnp.float32,
    )
    dist = (x_sq[None, :] - 2.0 * dot) + e_sq[:, None]

    idx = jnp.argmin(dist, axis=0).astype(jnp.int32)
    iota = jax.lax.broadcasted_iota(jnp.int32, (num_codes, 1), 0)
    onehot = (iota == idx[None, :]).astype(jnp.bfloat16)
    zq2 = jax.lax.dot_general(
        ecat, onehot, (((0,), (0,)), ((), ())),
        preferred_element_type=jnp.float32)
    zq = zq2[:d] + zq2[d:]

    zq_ref[0] = ze + (zq - ze)
    idx_ref[0, 0, :] = idx

    ones_col = jnp.ones((1, tblk), jnp.bfloat16)
    blk_counts = jax.lax.dot_general(
        ones_col, onehot, (((1,), (1,)), ((), ())),
        preferred_element_type=jnp.float32)
    blk_loss = jnp.full((128,), jnp.sum((ze - zq) ** 2), jnp.float32)

    @pl.when(i == 0)
    def _init():
        cnt_ref[0, :] = blk_counts[0]
        loss_ref[0, :] = blk_loss

    @pl.when(i > 0)
    def _accum():
        cnt_ref[0, :] += blk_counts[0]
        loss_ref[0, :] += blk_loss


def kernel(z_e, embedding):
    B, d, T = z_e.shape
    K = embedding.shape[0]
    Tblk = 8192
    tpb = T // Tblk
    grid = B * tpb

    emb_hi = embedding.astype(jnp.bfloat16)
    emb_lo = (embedding - emb_hi.astype(jnp.float32)).astype(jnp.bfloat16)
    emb_cat = jnp.concatenate([emb_hi, emb_lo], axis=1)

    zq_st, idx3, counts2, loss2 = pl.pallas_call(
        functools.partial(_vq_block_kernel, num_codes=K),
        grid=(grid,),
        in_specs=[
            pl.BlockSpec((1, d, Tblk), lambda i: (i // tpb, 0, i % tpb)),
            pl.BlockSpec((K, d), lambda i: (0, 0)),
            pl.BlockSpec((K, 2 * d), lambda i: (0, 0)),
        ],
        out_specs=[
            pl.BlockSpec((1, d, Tblk), lambda i: (i // tpb, 0, i % tpb)),
            pl.BlockSpec((1, 1, Tblk), lambda i: (i, 0, 0)),
            pl.BlockSpec((1, K), lambda i: (0, 0)),
            pl.BlockSpec((1, 128), lambda i: (0, 0)),
        ],
        out_shape=[
            jax.ShapeDtypeStruct((B, d, T), jnp.float32),
            jax.ShapeDtypeStruct((grid, 1, Tblk), jnp.int32),
            jax.ShapeDtypeStruct((1, K), jnp.float32),
            jax.ShapeDtypeStruct((1, 128), jnp.float32),
        ],
    )(z_e, embedding, emb_cat)

    indices = idx3.reshape(B, T)
    counts = counts2.reshape(K)
    commit_loss = 0.25 * (loss2[0, 0] / (B * d * T))
    probs = counts / jnp.maximum(counts.sum(), 1.0)
    perplexity = jnp.exp(-jnp.sum(probs * jnp.log(probs + 1e-10)))
    return (zq_st, commit_loss, indices, perplexity, counts)


# speedup vs baseline: 4.7948x; 1.0783x over previous
import functools

import jax
import jax.numpy as jnp
from jax.experimental import pallas as pl


def _vq_block_kernel(ze_ref, emb_ref, ecat_ref, em2_ref, zq_ref, idx_ref,
                     cnt_ref, loss_ref, *, num_codes):
    i = pl.program_id(0)
    ze = ze_ref[0]
    emb = emb_ref[...]
    ecat = ecat_ref[...]
    em2 = em2_ref[...]
    d = ze.shape[0]
    tblk = ze.shape[1]

    e_sq = jnp.sum(emb * emb, axis=1)
    x_sq = jnp.sum(ze * ze, axis=0)
    dotm2 = jax.lax.dot_general(
        em2, ze.astype(jnp.bfloat16), (((1,), (0,)), ((), ())),
        preferred_element_type=jnp.float32,
    )
    dist = (x_sq[None, :] + dotm2) + e_sq[:, None]

    idx = jnp.argmin(dist, axis=0).astype(jnp.int32)
    iota = jax.lax.broadcasted_iota(jnp.int32, (num_codes, 1), 0)
    onehot = (iota == idx[None, :]).astype(jnp.bfloat16)
    zq2 = jax.lax.dot_general(
        ecat, onehot, (((0,), (0,)), ((), ())),
        preferred_element_type=jnp.float32)
    zq = zq2[:d] + zq2[d:]

    zq_ref[0] = zq
    idx_ref[0, 0, :] = idx

    ones_col = jnp.ones((1, tblk), jnp.bfloat16)
    blk_counts = jax.lax.dot_general(
        ones_col, onehot, (((1,), (1,)), ((), ())),
        preferred_element_type=jnp.float32)
    blk_loss = jnp.full((128,), jnp.sum((ze - zq) ** 2), jnp.float32)

    @pl.when(i == 0)
    def _init():
        cnt_ref[0, :] = blk_counts[0]
        loss_ref[0, :] = blk_loss

    @pl.when(i > 0)
    def _accum():
        cnt_ref[0, :] += blk_counts[0]
        loss_ref[0, :] += blk_loss


def kernel(z_e, embedding):
    B, d, T = z_e.shape
    K = embedding.shape[0]
    Tblk = 8192
    tpb = T // Tblk
    grid = B * tpb

    emb_hi = embedding.astype(jnp.bfloat16)
    emb_lo = (embedding - emb_hi.astype(jnp.float32)).astype(jnp.bfloat16)
    emb_cat = jnp.concatenate([emb_hi, emb_lo], axis=1)
    emb_m2 = (jnp.float32(-2.0) * emb_hi.astype(jnp.float32)).astype(jnp.bfloat16)

    zq_st, idx3, counts2, loss2 = pl.pallas_call(
        functools.partial(_vq_block_kernel, num_codes=K),
        grid=(grid,),
        in_specs=[
            pl.BlockSpec((1, d, Tblk), lambda i: (i // tpb, 0, i % tpb)),
            pl.BlockSpec((K, d), lambda i: (0, 0)),
            pl.BlockSpec((K, 2 * d), lambda i: (0, 0)),
            pl.BlockSpec((K, d), lambda i: (0, 0)),
        ],
        out_specs=[
            pl.BlockSpec((1, d, Tblk), lambda i: (i // tpb, 0, i % tpb)),
            pl.BlockSpec((1, 1, Tblk), lambda i: (i, 0, 0)),
            pl.BlockSpec((1, K), lambda i: (0, 0)),
            pl.BlockSpec((1, 128), lambda i: (0, 0)),
        ],
        out_shape=[
            jax.ShapeDtypeStruct((B, d, T), jnp.float32),
            jax.ShapeDtypeStruct((grid, 1, Tblk), jnp.int32),
            jax.ShapeDtypeStruct((1, K), jnp.float32),
            jax.ShapeDtypeStruct((1, 128), jnp.float32),
        ],
    )(z_e, embedding, emb_cat, emb_m2)

    indices = idx3.reshape(B, T)
    counts = counts2.reshape(K)
    commit_loss = 0.25 * (loss2[0, 0] / (B * d * T))
    probs = counts / jnp.maximum(counts.sum(), 1.0)
    perplexity = jnp.exp(-jnp.sum(probs * jnp.log(probs + 1e-10)))
    return (zq_st, commit_loss, indices, perplexity, counts)
